# Initial kernel scaffold; baseline (speedup 1.0000x reference)
import jax
import jax.numpy as jnp
from jax.experimental import pallas as pl


def kernel(hidden_states, weight):
    raise NotImplementedError("write your pallas kernel here")


# speedup vs baseline: 1.1502x; 1.1502x over previous
import jax
import jax.numpy as jnp
from jax.experimental import pallas as pl
from jax.experimental.pallas import tpu as pltpu

_E = 64
_TOPK = 8
_BLK = 512


def _gate_kernel(hs_ref, wt_ref, idx_ref, w_ref):
    hs = hs_ref[...]
    wt = wt_ref[...]
    logits = jnp.dot(hs, wt, preferred_element_type=jnp.float32)
    rowmax = jnp.max(logits, axis=-1, keepdims=True)
    p = jnp.exp(logits - rowmax)
    z = jnp.sum(p, axis=-1, keepdims=True)
    scores = p / z
    b = scores.shape[0]
    iota = jax.lax.broadcasted_iota(jnp.int32, (b, _E), 1)
    work = scores
    vals = []
    idxs = []
    for _ in range(_TOPK):
        m = jnp.max(work, axis=-1, keepdims=True)
        im = jnp.min(jnp.where(work == m, iota, _E), axis=-1, keepdims=True)
        vals.append(m)
        idxs.append(im)
        work = jnp.where(iota == im, -1.0, work)
    v = jnp.concatenate(vals, axis=-1)
    i = jnp.concatenate(idxs, axis=-1)
    denom = jnp.sum(v, axis=-1, keepdims=True) + 1e-20
    idx_ref[...] = i
    w_ref[...] = v / denom


def kernel(hidden_states, weight):
    bsz, seq, h = hidden_states.shape
    t = bsz * seq
    hs = hidden_states.reshape(t, h)
    wt = weight.T

    idx, w = pl.pallas_call(
        _gate_kernel,
        grid=(t // _BLK,),
        in_specs=[
            pl.BlockSpec((_BLK, h), lambda i: (i, 0)),
            pl.BlockSpec((h, _E), lambda i: (0, 0)),
        ],
        out_specs=[
            pl.BlockSpec((_BLK, _TOPK), lambda i: (i, 0)),
            pl.BlockSpec((_BLK, _TOPK), lambda i: (i, 0)),
        ],
        out_shape=[
            jax.ShapeDtypeStruct((t, _TOPK), jnp.int32),
            jax.ShapeDtypeStruct((t, _TOPK), jnp.float32),
        ],
    )(hs, wt)

    return (idx.reshape(t, _TOPK), w.reshape(t, _TOPK))


# speedup vs baseline: 1.3068x; 1.1361x over previous
import jax
import jax.numpy as jnp
from jax.experimental import pallas as pl
from jax.experimental.pallas import tpu as pltpu

_E = 64
_TOPK = 8
_BLK = 1024


def _gate_kernel(hs_ref, wt_ref, idx_ref, w_ref):
    hs = hs_ref[...]
    wt = wt_ref[...]
    logits = jnp.dot(hs, wt, preferred_element_type=jnp.float32)
    rowmax = jnp.max(logits, axis=-1, keepdims=True)
    p = jnp.exp(logits - rowmax)
    z = jnp.sum(p, axis=-1, keepdims=True)
    scores = p / z
    b = scores.shape[0]
    iota = jax.lax.broadcasted_iota(jnp.int32, (b, _E), 1)
    work = scores
    vals = []
    idxs = []
    for _ in range(_TOPK):
        m = jnp.max(work, axis=-1, keepdims=True)
        im = jnp.min(jnp.where(work == m, iota, _E), axis=-1, keepdims=True)
        vals.append(m)
        idxs.append(im)
        work = jnp.where(iota == im, -1.0, work)
    v = jnp.concatenate(vals, axis=-1)
    i = jnp.concatenate(idxs, axis=-1)
    denom = jnp.sum(v, axis=-1, keepdims=True) + 1e-20
    idx_ref[...] = i
    w_ref[...] = v / denom


def kernel(hidden_states, weight):
    bsz, seq, h = hidden_states.shape
    t = bsz * seq
    hs = hidden_states.reshape(t, h)
    wt = weight.T

    idx, w = pl.pallas_call(
        _gate_kernel,
        grid=(t // _BLK,),
        in_specs=[
            pl.BlockSpec((_BLK, h), lambda i: (i, 0)),
            pl.BlockSpec((h, _E), lambda i: (0, 0)),
        ],
        out_specs=[
            pl.BlockSpec((_BLK, _TOPK), lambda i: (i, 0)),
            pl.BlockSpec((_BLK, _TOPK), lambda i: (i, 0)),
        ],
        out_shape=[
            jax.ShapeDtypeStruct((t, _TOPK), jnp.int32),
            jax.ShapeDtypeStruct((t, _TOPK), jnp.float32),
        ],
    )(hs, wt)

    return (idx.reshape(t, _TOPK), w.reshape(t, _TOPK))


# speedup vs baseline: 1.4092x; 1.0784x over previous
import jax
import jax.numpy as jnp
from jax.experimental import pallas as pl
from jax.experimental.pallas import tpu as pltpu

_E = 64
_TOPK = 8
_BLK = 1024


def _gate_kernel(hs_ref, wt_ref, idx_ref, w_ref):
    hs = hs_ref[...]
    wt = wt_ref[...]
    logits = jnp.dot(hs, wt, preferred_element_type=jnp.float32)
    rowmax = jnp.max(logits, axis=-1, keepdims=True)
    p = jnp.exp(logits - rowmax)
    b = p.shape[0]
    iota = jax.lax.broadcasted_iota(jnp.int32, (b, _E), 1)
    bits = jax.lax.bitcast_convert_type(p, jnp.int32)
    work = (bits & ~0x3F) | (_E - 1 - iota)
    vals = []
    idxs = []
    for _ in range(_TOPK):
        m = jnp.max(work, axis=-1, keepdims=True)
        idxs.append(_E - 1 - (m & 0x3F))
        vals.append(m & ~0x3F)
        work = jnp.where(work == m, 0, work)
    v = jax.lax.bitcast_convert_type(
        jnp.concatenate(vals, axis=-1), jnp.float32)
    i = jnp.concatenate(idxs, axis=-1)
    denom = jnp.sum(v, axis=-1, keepdims=True) + 1e-20
    idx_ref[...] = i
    w_ref[...] = v / denom


def kernel(hidden_states, weight):
    bsz, seq, h = hidden_states.shape
    t = bsz * seq
    hs = hidden_states.reshape(t, h)
    wt = weight.T

    idx, w = pl.pallas_call(
        _gate_kernel,
        grid=(t // _BLK,),
        in_specs=[
            pl.BlockSpec((_BLK, h), lambda i: (i, 0)),
            pl.BlockSpec((h, _E), lambda i: (0, 0)),
        ],
        out_specs=[
            pl.BlockSpec((_BLK, _TOPK), lambda i: (i, 0)),
            pl.BlockSpec((_BLK, _TOPK), lambda i: (i, 0)),
        ],
        out_shape=[
            jax.ShapeDtypeStruct((t, _TOPK), jnp.int32),
            jax.ShapeDtypeStruct((t, _TOPK), jnp.float32),
        ],
    )(hs, wt)

    return (idx.reshape(t, _TOPK), w.reshape(t, _TOPK))
